# baseline (speedup 1.0000x reference)
import functools

import jax
import jax.numpy as jnp
from jax import lax
from jax.experimental import pallas as pl
from jax.experimental.pallas import tpu as pltpu
from jax.experimental.pallas import tpu_sc as plsc


def _tc_body(zp_ref, w1_ref, b1_ref, w2_ref, b2_ref, cbt_ref, cb_ref,
             w3_ref, b3_ref, idx_ref, lut_ref, cc_ref):
    i = pl.program_id(0)

    @pl.when(i == 0)
    def _():
        cbt = cbt_ref[...]
        cc_ref[...] = jnp.sum(cbt * cbt, axis=0, keepdims=True)
        lut_ref[...] = (jnp.dot(cb_ref[...], w3_ref[...],
                                preferred_element_type=jnp.float32)
                        + b3_ref[...])

    h1 = jnp.dot(zp_ref[...], w1_ref[...], preferred_element_type=jnp.float32)
    a1 = jnp.maximum(h1 + b1_ref[...], 0.0)
    h2 = jnp.dot(a1, w2_ref[...], preferred_element_type=jnp.float32)
    a2 = jnp.maximum(h2 + b2_ref[...], 0.0)

    xc = jnp.dot(a2, cbt_ref[...], preferred_element_type=jnp.float32)
    xx = jnp.sum(a2 * a2, axis=1, keepdims=True)
    s = (xx - 2.0 * xc) + cc_ref[...]

    m = jnp.min(s, axis=1, keepdims=True)
    iota = lax.broadcasted_iota(jnp.int32, s.shape, 1)
    cand = jnp.where(s == m, iota, s.shape[1])
    idx_ref[...] = jnp.min(cand, axis=1, keepdims=True)


def _encode_and_lut(zp, w1p, b1r, w2, b2r, cbt, cb, w3p, b3r, *, rows):
    n = zp.shape[0]
    k = cb.shape[0]
    nbp = w3p.shape[1]
    grid = (n // rows,)
    return pl.pallas_call(
        _tc_body,
        grid=grid,
        in_specs=[
            pl.BlockSpec((rows, zp.shape[1]), lambda i: (i, 0)),
            pl.BlockSpec(w1p.shape, lambda i: (0, 0)),
            pl.BlockSpec(b1r.shape, lambda i: (0, 0)),
            pl.BlockSpec(w2.shape, lambda i: (0, 0)),
            pl.BlockSpec(b2r.shape, lambda i: (0, 0)),
            pl.BlockSpec(cbt.shape, lambda i: (0, 0)),
            pl.BlockSpec(cb.shape, lambda i: (0, 0)),
            pl.BlockSpec(w3p.shape, lambda i: (0, 0)),
            pl.BlockSpec(b3r.shape, lambda i: (0, 0)),
        ],
        out_specs=[
            pl.BlockSpec((rows, 1), lambda i: (i, 0)),
            pl.BlockSpec((k, nbp), lambda i: (0, 0)),
        ],
        out_shape=[
            jax.ShapeDtypeStruct((n, 1), jnp.int32),
            jax.ShapeDtypeStruct((k, nbp), jnp.float32),
        ],
        scratch_shapes=[pltpu.VMEM((1, k), jnp.float32)],
    )(zp, w1p, b1r, w2, b2r, cbt, cb, w3p, b3r)


def _sc_gather(lutb, idx2):
    n = idx2.shape[0] * 128
    nbp = lutb.shape[1]
    rows_per_tile = n // 32
    idx_rows = rows_per_tile // 128

    mesh = plsc.VectorSubcoreMesh(core_axis_name="c", subcore_axis_name="s")

    @functools.partial(
        pl.kernel,
        mesh=mesh,
        out_type=jax.ShapeDtypeStruct((n, nbp), jnp.float32),
        scratch_types=[
            pltpu.VMEM((idx_rows, 128), jnp.int32),
            pltpu.VMEM((rows_per_tile // 2, nbp), jnp.float32),
            pltpu.SemaphoreType.DMA,
        ],
    )
    def k(lut_hbm, idx_hbm, out_hbm, idx_v, rows_v, sem):
        wid = lax.axis_index("s") * 2 + lax.axis_index("c")
        pltpu.sync_copy(idx_hbm.at[pl.ds(wid * idx_rows, idx_rows)], idx_v)
        for half in range(2):
            copies = [
                pltpu.async_copy(lut_hbm.at[idx_v.at[half * 4 + j]],
                                 rows_v.at[pl.ds(j * 128, 128)], sem)
                for j in range(4)
            ]
            for c in copies:
                c.wait()
            pltpu.sync_copy(
                rows_v,
                out_hbm.at[pl.ds(wid * rows_per_tile + half * 512, 512)])

    return k(lutb, idx2)


def kernel(z, W1, b1, W2, b2, W3, b3, codebook):
    bsz, s, f = z.shape
    h = W1.shape[1]
    nb = W3.shape[1]
    n = bsz * s

    zf = z.reshape(n, f)
    zp = jnp.pad(zf, ((0, 0), (0, 8 - f)))
    w1p = jnp.pad(W1, ((0, 8 - f), (0, 0)))
    w3p = jnp.pad(W3, ((0, 0), (0, 128 - nb)))
    b3r = jnp.pad(b3, ((0, 128 - nb),)).reshape(1, 128)
    b1r = b1.reshape(1, h)
    b2r = b2.reshape(1, h)
    cbt = codebook.T

    idx_n1, lutb = _encode_and_lut(zp, w1p, b1r, w2=W2, b2r=b2r, cbt=cbt,
                                   cb=codebook, w3p=w3p, b3r=b3r, rows=256)
    idx2 = idx_n1.reshape(n // 128, 128)
    gathered = _sc_gather(lutb, idx2)
    return gathered[:, :nb].reshape(bsz, s * nb)


# speedup vs baseline: 1.0447x; 1.0447x over previous
import functools

import jax
import jax.numpy as jnp
from jax import lax
from jax.experimental import pallas as pl
from jax.experimental.pallas import tpu as pltpu
from jax.experimental.pallas import tpu_sc as plsc


_CH = 128


def _tc_body(zt_ref, w1t_ref, b1c_ref, w2t_ref, b2c_ref, cb_ref,
             w3_ref, b3_ref, idx_ref, lut_ref, cc_ref):
    i = pl.program_id(0)
    k = cb_ref.shape[0]

    @pl.when(i == 0)
    def _():
        cb = cb_ref[...]
        cc_ref[...] = jnp.sum(cb * cb, axis=1, keepdims=True)
        lut_ref[...] = (jnp.dot(cb, w3_ref[...],
                                preferred_element_type=jnp.float32)
                        + b3_ref[...])

    h1 = jnp.dot(w1t_ref[...], zt_ref[...], preferred_element_type=jnp.float32)
    a1 = jnp.maximum(h1 + b1c_ref[...], 0.0)
    h2 = jnp.dot(w2t_ref[...], a1, preferred_element_type=jnp.float32)
    a2 = jnp.maximum(h2 + b2c_ref[...], 0.0)

    m2 = jnp.dot(cb_ref[...], a2 * -2.0, preferred_element_type=jnp.float32)
    xx = jnp.sum(a2 * a2, axis=0, keepdims=True)
    s = (xx + m2) + cc_ref[...]

    acc_v = s[0:_CH, :]
    acc_c = jnp.zeros(acc_v.shape, jnp.int32)
    for j in range(1, k // _CH):
        sc = s[j * _CH:(j + 1) * _CH, :]
        lt = sc < acc_v
        acc_v = jnp.where(lt, sc, acc_v)
        acc_c = jnp.where(lt, j, acc_c)
    m = jnp.min(acc_v, axis=0, keepdims=True)
    kpos = acc_c * _CH + lax.broadcasted_iota(jnp.int32, acc_c.shape, 0)
    cand = jnp.where(acc_v == m, kpos, k)
    idx = jnp.min(cand, axis=0, keepdims=True)
    idx_ref[...] = idx.reshape(idx_ref.shape)


def _encode_and_lut(zt, w1t, b1c, w2t, b2c, cb, w3p, b3r, *, rows):
    n = zt.shape[1]
    k = cb.shape[0]
    nbp = w3p.shape[1]
    grid = (n // rows,)
    return pl.pallas_call(
        _tc_body,
        grid=grid,
        in_specs=[
            pl.BlockSpec((zt.shape[0], rows), lambda i: (0, i)),
            pl.BlockSpec(w1t.shape, lambda i: (0, 0)),
            pl.BlockSpec(b1c.shape, lambda i: (0, 0)),
            pl.BlockSpec(w2t.shape, lambda i: (0, 0)),
            pl.BlockSpec(b2c.shape, lambda i: (0, 0)),
            pl.BlockSpec(cb.shape, lambda i: (0, 0)),
            pl.BlockSpec(w3p.shape, lambda i: (0, 0)),
            pl.BlockSpec(b3r.shape, lambda i: (0, 0)),
        ],
        out_specs=[
            pl.BlockSpec((1, 1, rows), lambda i: (i, 0, 0)),
            pl.BlockSpec((k, nbp), lambda i: (0, 0)),
        ],
        out_shape=[
            jax.ShapeDtypeStruct((n // rows, 1, rows), jnp.int32),
            jax.ShapeDtypeStruct((k, nbp), jnp.float32),
        ],
        scratch_shapes=[pltpu.VMEM((k, 1), jnp.float32)],
    )(zt, w1t, b1c, w2t, b2c, cb, w3p, b3r)


def _sc_gather(lutb, idx2):
    n = idx2.shape[0] * 128
    nbp = lutb.shape[1]
    rows_per_tile = n // 32
    idx_rows = rows_per_tile // 128

    mesh = plsc.VectorSubcoreMesh(core_axis_name="c", subcore_axis_name="s")

    @functools.partial(
        pl.kernel,
        mesh=mesh,
        out_type=jax.ShapeDtypeStruct((n, nbp), jnp.float32),
        scratch_types=[
            pltpu.VMEM((idx_rows, 128), jnp.int32),
            pltpu.VMEM((rows_per_tile // 2, nbp), jnp.float32),
            pltpu.SemaphoreType.DMA,
        ],
    )
    def k(lut_hbm, idx_hbm, out_hbm, idx_v, rows_v, sem):
        wid = lax.axis_index("s") * 2 + lax.axis_index("c")
        pltpu.sync_copy(idx_hbm.at[pl.ds(wid * idx_rows, idx_rows)], idx_v)
        for half in range(2):
            copies = [
                pltpu.async_copy(lut_hbm.at[idx_v.at[half * 4 + j]],
                                 rows_v.at[pl.ds(j * 128, 128)], sem)
                for j in range(4)
            ]
            for c in copies:
                c.wait()
            pltpu.sync_copy(
                rows_v,
                out_hbm.at[pl.ds(wid * rows_per_tile + half * 512, 512)])

    return k(lutb, idx2)


def kernel(z, W1, b1, W2, b2, W3, b3, codebook):
    bsz, s, f = z.shape
    h = W1.shape[1]
    nb = W3.shape[1]
    n = bsz * s

    zt = jnp.pad(z.reshape(n, f), ((0, 0), (0, 8 - f))).T
    w1t = jnp.pad(W1, ((0, 8 - f), (0, 0))).T
    w3p = jnp.pad(W3, ((0, 0), (0, 128 - nb)))
    b3r = jnp.pad(b3, ((0, 128 - nb),)).reshape(1, 128)
    b1c = b1.reshape(h, 1)
    b2c = b2.reshape(h, 1)

    idx_n1, lutb = _encode_and_lut(zt, w1t, b1c, w2t=W2.T, b2c=b2c,
                                   cb=codebook, w3p=w3p, b3r=b3r, rows=256)
    idx2 = idx_n1.reshape(n // 128, 128)
    gathered = _sc_gather(lutb, idx2)
    return gathered[:, :nb].reshape(bsz, s * nb)


# speedup vs baseline: 4.0840x; 3.9092x over previous
import functools

import jax
import jax.numpy as jnp
from jax import lax
from jax.experimental import pallas as pl
from jax.experimental.pallas import tpu as pltpu
from jax.experimental.pallas import tpu_sc as plsc


_CH = 128


def _tc_body(zt_ref, w1t_ref, b1c_ref, w2t_ref, b2c_ref, cb_ref,
             w3_ref, b3_ref, idx_ref, lut_ref, cc_ref):
    i = pl.program_id(0)
    k = cb_ref.shape[0]

    @pl.when(i == 0)
    def _():
        cb = cb_ref[...]
        cc_ref[...] = jnp.sum(cb * cb, axis=1, keepdims=True)
        lut_ref[...] = (jnp.dot(cb, w3_ref[...],
                                preferred_element_type=jnp.float32)
                        + b3_ref[...])

    h1 = jnp.dot(w1t_ref[...], zt_ref[...], preferred_element_type=jnp.float32)
    a1 = jnp.maximum(h1 + b1c_ref[...], 0.0)
    h2 = jnp.dot(w2t_ref[...], a1, preferred_element_type=jnp.float32)
    a2 = jnp.maximum(h2 + b2c_ref[...], 0.0)

    m2 = jnp.dot(cb_ref[...], a2 * -2.0, preferred_element_type=jnp.float32)
    xx = jnp.sum(a2 * a2, axis=0, keepdims=True)
    s = (xx + m2) + cc_ref[...]

    acc_v = s[0:_CH, :]
    acc_c = jnp.zeros(acc_v.shape, jnp.int32)
    for j in range(1, k // _CH):
        sc = s[j * _CH:(j + 1) * _CH, :]
        lt = sc < acc_v
        acc_v = jnp.where(lt, sc, acc_v)
        acc_c = jnp.where(lt, j, acc_c)
    m = jnp.min(acc_v, axis=0, keepdims=True)
    kpos = acc_c * _CH + lax.broadcasted_iota(jnp.int32, acc_c.shape, 0)
    cand = jnp.where(acc_v == m, kpos, k)
    idx = jnp.min(cand, axis=0, keepdims=True)
    idx_ref[...] = idx.reshape(idx_ref.shape)


def _encode_and_lut(zt, w1t, b1c, w2t, b2c, cb, w3p, b3r, *, rows):
    n = zt.shape[1]
    k = cb.shape[0]
    nbp = w3p.shape[1]
    grid = (n // rows,)
    return pl.pallas_call(
        _tc_body,
        grid=grid,
        in_specs=[
            pl.BlockSpec((zt.shape[0], rows), lambda i: (0, i)),
            pl.BlockSpec(w1t.shape, lambda i: (0, 0)),
            pl.BlockSpec(b1c.shape, lambda i: (0, 0)),
            pl.BlockSpec(w2t.shape, lambda i: (0, 0)),
            pl.BlockSpec(b2c.shape, lambda i: (0, 0)),
            pl.BlockSpec(cb.shape, lambda i: (0, 0)),
            pl.BlockSpec(w3p.shape, lambda i: (0, 0)),
            pl.BlockSpec(b3r.shape, lambda i: (0, 0)),
        ],
        out_specs=[
            pl.BlockSpec((1, 1, rows), lambda i: (i, 0, 0)),
            pl.BlockSpec((k, nbp), lambda i: (0, 0)),
        ],
        out_shape=[
            jax.ShapeDtypeStruct((n // rows, 1, rows), jnp.int32),
            jax.ShapeDtypeStruct((k, nbp), jnp.float32),
        ],
        scratch_shapes=[pltpu.VMEM((k, 1), jnp.float32)],
    )(zt, w1t, b1c, w2t, b2c, cb, w3p, b3r)


def _sc_gather(lutp, idx2, nb):
    n = idx2.shape[0] * 128
    rows_per_tile = n // 32
    idx_rows = rows_per_tile // 128

    mesh = plsc.VectorSubcoreMesh(core_axis_name="c", subcore_axis_name="s")

    @functools.partial(
        pl.kernel,
        mesh=mesh,
        compiler_params=pltpu.CompilerParams(needs_layout_passes=False),
        out_type=jax.ShapeDtypeStruct((32, nb, rows_per_tile), jnp.float32),
        scratch_types=[
            pltpu.VMEM(lutp.shape, jnp.float32),
            pltpu.VMEM((idx_rows, 128), jnp.int32),
            pltpu.VMEM((nb, rows_per_tile), jnp.float32),
            pltpu.SemaphoreType.DMA,
        ],
    )
    def k(lut_hbm, idx_hbm, out_hbm, lut_v, idx_v, out_v, sem):
        wid = lax.axis_index("s") * 2 + lax.axis_index("c")
        cp = pltpu.async_copy(lut_hbm, lut_v, sem)
        pltpu.sync_copy(idx_hbm.at[pl.ds(wid * idx_rows, idx_rows)], idx_v)
        cp.wait()

        def row_body(row, carry):
            for g in range(8):
                iv = idx_v[row, pl.ds(g * 16, 16)]
                ivd = lax.shift_right_logical(iv, 3)
                ivm = lax.shift_left(jnp.bitwise_and(iv, 7), 4)
                for j in range(nb):
                    vals = plsc.load_gather(lut_v, [ivd, ivm + j])
                    out_v[j, pl.ds(row * 128 + g * 16, 16)] = vals
            return carry

        lax.fori_loop(0, idx_rows, row_body, 0)
        pltpu.sync_copy(out_v, out_hbm.at[wid])

    return k(lutp, idx2)


def kernel(z, W1, b1, W2, b2, W3, b3, codebook):
    bsz, s, f = z.shape
    h = W1.shape[1]
    nb = W3.shape[1]
    n = bsz * s

    zt = jnp.pad(z.reshape(n, f), ((0, 0), (0, 8 - f))).T
    w1t = jnp.pad(W1, ((0, 8 - f), (0, 0))).T
    w3p = jnp.pad(W3, ((0, 0), (0, 16 - nb)))
    b3r = jnp.pad(b3, ((0, 16 - nb),)).reshape(1, 16)
    b1c = b1.reshape(h, 1)
    b2c = b2.reshape(h, 1)

    idx_n1, lutb = _encode_and_lut(zt, w1t, b1c, w2t=W2.T, b2c=b2c,
                                   cb=codebook, w3p=w3p, b3r=b3r, rows=256)
    idx2 = idx_n1.reshape(n // 128, 128)
    lutp = lutb.reshape(-1, 128)
    gathered = _sc_gather(lutp, idx2, nb)
    return gathered.transpose(0, 2, 1).reshape(bsz, s * nb)


# speedup vs baseline: 5.5783x; 1.3659x over previous
import functools

import jax
import jax.numpy as jnp
from jax import lax
from jax.experimental import pallas as pl
from jax.experimental.pallas import tpu as pltpu
from jax.experimental.pallas import tpu_sc as plsc


_CH = 128


def _tc_body(zt_ref, w1t_ref, b1c_ref, w2t_ref, b2c_ref, cb_ref,
             w3_ref, b3_ref, idx_ref, lut_ref, aug_ref):
    i = pl.program_id(0)
    k = cb_ref.shape[0]
    h = cb_ref.shape[1]

    @pl.when(i == 0)
    def _():
        cb = cb_ref[...]
        lut_ref[...] = (jnp.dot(cb, w3_ref[...],
                                preferred_element_type=jnp.float32)
                        + b3_ref[...])
        cc = jnp.sum(cb * cb, axis=1, keepdims=True)
        cc1 = cc.astype(jnp.bfloat16).astype(jnp.float32)
        r1 = cc - cc1
        cc2 = r1.astype(jnp.bfloat16).astype(jnp.float32)
        cc3 = (r1 - cc2).astype(jnp.bfloat16).astype(jnp.float32)
        lane = lax.broadcasted_iota(jnp.int32, (k, h), 1)
        hi = jnp.where(lane == 0, cc1,
                       jnp.where(lane == 1, cc2,
                                 jnp.where(lane == 2, cc3, 0.0)))
        aug_ref[:, 0:h] = cb
        aug_ref[:, h:2 * h] = hi

    h1 = jnp.dot(w1t_ref[...], zt_ref[...], preferred_element_type=jnp.float32)
    a1 = jnp.maximum(h1 + b1c_ref[...], 0.0)
    h2 = jnp.dot(w2t_ref[...], a1, preferred_element_type=jnp.float32)
    a2 = jnp.maximum(h2 + b2c_ref[...], 0.0)

    sub = lax.broadcasted_iota(jnp.int32, a2.shape, 0)
    ones3 = jnp.where(sub < 3, 1.0, 0.0)
    rhs = jnp.concatenate([a2 * -2.0, ones3], axis=0)
    s = jnp.dot(aug_ref[...], rhs, preferred_element_type=jnp.float32)

    acc_v = s[0:_CH, :]
    acc_c = jnp.zeros(acc_v.shape, jnp.int32)
    for j in range(1, k // _CH):
        sc = s[j * _CH:(j + 1) * _CH, :]
        lt = sc < acc_v
        acc_v = jnp.where(lt, sc, acc_v)
        acc_c = jnp.where(lt, j, acc_c)
    m = jnp.min(acc_v, axis=0, keepdims=True)
    kpos = acc_c * _CH + lax.broadcasted_iota(jnp.int32, acc_c.shape, 0)
    cand = jnp.where(acc_v == m, kpos, k)
    idx = jnp.min(cand, axis=0, keepdims=True)
    idx_ref[...] = idx.reshape(idx_ref.shape)


def _encode_and_lut(zt, w1t, b1c, w2t, b2c, cb, w3p, b3r, *, rows):
    n = zt.shape[1]
    k = cb.shape[0]
    nbp = w3p.shape[1]
    grid = (n // rows,)
    return pl.pallas_call(
        _tc_body,
        grid=grid,
        in_specs=[
            pl.BlockSpec((zt.shape[0], rows), lambda i: (0, i)),
            pl.BlockSpec(w1t.shape, lambda i: (0, 0)),
            pl.BlockSpec(b1c.shape, lambda i: (0, 0)),
            pl.BlockSpec(w2t.shape, lambda i: (0, 0)),
            pl.BlockSpec(b2c.shape, lambda i: (0, 0)),
            pl.BlockSpec(cb.shape, lambda i: (0, 0)),
            pl.BlockSpec(w3p.shape, lambda i: (0, 0)),
            pl.BlockSpec(b3r.shape, lambda i: (0, 0)),
        ],
        out_specs=[
            pl.BlockSpec((1, 1, rows), lambda i: (i, 0, 0)),
            pl.BlockSpec((k, nbp), lambda i: (0, 0)),
        ],
        out_shape=[
            jax.ShapeDtypeStruct((n // rows, 1, rows), jnp.int32),
            jax.ShapeDtypeStruct((k, nbp), jnp.float32),
        ],
        scratch_shapes=[pltpu.VMEM((k, 2 * cb.shape[1]), jnp.float32)],
    )(zt, w1t, b1c, w2t, b2c, cb, w3p, b3r)


def _sc_gather(lutp, idx2, nb):
    n = idx2.shape[0] * 128
    rows_per_tile = n // 32
    idx_rows = rows_per_tile // 128

    mesh = plsc.VectorSubcoreMesh(core_axis_name="c", subcore_axis_name="s")

    @functools.partial(
        pl.kernel,
        mesh=mesh,
        compiler_params=pltpu.CompilerParams(needs_layout_passes=False),
        out_type=jax.ShapeDtypeStruct((32, nb, rows_per_tile), jnp.float32),
        scratch_types=[
            pltpu.VMEM(lutp.shape, jnp.float32),
            pltpu.VMEM((idx_rows, 128), jnp.int32),
            pltpu.VMEM((nb, rows_per_tile), jnp.float32),
            pltpu.SemaphoreType.DMA,
        ],
    )
    def k(lut_hbm, idx_hbm, out_hbm, lut_v, idx_v, out_v, sem):
        wid = lax.axis_index("s") * 2 + lax.axis_index("c")
        cp = pltpu.async_copy(lut_hbm, lut_v, sem)
        pltpu.sync_copy(idx_hbm.at[pl.ds(wid * idx_rows, idx_rows)], idx_v)
        cp.wait()

        def row_body(row, carry):
            for g in range(8):
                iv = idx_v[row, pl.ds(g * 16, 16)]
                ivd = lax.shift_right_logical(iv, 3)
                ivm = lax.shift_left(jnp.bitwise_and(iv, 7), 4)
                for j in range(nb):
                    vals = plsc.load_gather(lut_v, [ivd, ivm + j])
                    out_v[j, pl.ds(row * 128 + g * 16, 16)] = vals
            return carry

        lax.fori_loop(0, idx_rows, row_body, 0)
        pltpu.sync_copy(out_v, out_hbm.at[wid])

    return k(lutp, idx2)


def kernel(z, W1, b1, W2, b2, W3, b3, codebook):
    bsz, s, f = z.shape
    h = W1.shape[1]
    nb = W3.shape[1]
    n = bsz * s

    zt = jnp.pad(z.reshape(n, f), ((0, 0), (0, 8 - f))).T
    w1t = jnp.pad(W1, ((0, 8 - f), (0, 0))).T
    w3p = jnp.pad(W3, ((0, 0), (0, 16 - nb)))
    b3r = jnp.pad(b3, ((0, 16 - nb),)).reshape(1, 16)
    b1c = b1.reshape(h, 1)
    b2c = b2.reshape(h, 1)

    idx_n1, lutb = _encode_and_lut(zt, w1t, b1c, w2t=W2.T, b2c=b2c,
                                   cb=codebook, w3p=w3p, b3r=b3r, rows=512)
    idx2 = idx_n1.reshape(n // 128, 128)
    lutp = lutb.reshape(-1, 128)
    gathered = _sc_gather(lutp, idx2, nb)
    return gathered.transpose(0, 2, 1).reshape(bsz, s * nb)


# speedup vs baseline: 5.7140x; 1.0243x over previous
import functools

import jax
import jax.numpy as jnp
from jax import lax
from jax.experimental import pallas as pl
from jax.experimental.pallas import tpu as pltpu
from jax.experimental.pallas import tpu_sc as plsc


_CH = 16


def _tc_body(zt_ref, w1t_ref, b1c_ref, w2t_ref, b2c_ref, cb_ref,
             w3_ref, b3_ref, idx_ref, lut_ref, aug_ref):
    i = pl.program_id(0)
    k = cb_ref.shape[0]
    h = cb_ref.shape[1]

    @pl.when(i == 0)
    def _():
        cb = cb_ref[...]
        lut_ref[...] = (jnp.dot(cb, w3_ref[...],
                                preferred_element_type=jnp.float32)
                        + b3_ref[...])
        cc = jnp.sum(cb * cb, axis=1, keepdims=True)
        cc1 = cc.astype(jnp.bfloat16).astype(jnp.float32)
        r1 = cc - cc1
        cc2 = r1.astype(jnp.bfloat16).astype(jnp.float32)
        cc3 = (r1 - cc2).astype(jnp.bfloat16).astype(jnp.float32)
        lane = lax.broadcasted_iota(jnp.int32, (k, h), 1)
        hi = jnp.where(lane == 0, cc1,
                       jnp.where(lane == 1, cc2,
                                 jnp.where(lane == 2, cc3, 0.0)))
        aug_ref[:, 0:h] = cb.astype(jnp.bfloat16)
        aug_ref[:, h:2 * h] = hi.astype(jnp.bfloat16)

    h1 = jnp.dot(w1t_ref[...], zt_ref[...], preferred_element_type=jnp.float32)
    a1 = jnp.maximum(h1 + b1c_ref[...], 0.0)
    h2 = jnp.dot(w2t_ref[...], a1, preferred_element_type=jnp.float32)
    a2 = jnp.maximum(h2 + b2c_ref[...], 0.0)

    sub = lax.broadcasted_iota(jnp.int32, a2.shape, 0)
    ones3 = jnp.where(sub < 3, 1.0, 0.0)
    rhs = jnp.concatenate([a2 * -2.0, ones3], axis=0)
    s = jnp.dot(aug_ref[...], rhs.astype(jnp.bfloat16),
                preferred_element_type=jnp.float32)

    acc_v = s[0:_CH, :]
    acc_c = jnp.zeros(acc_v.shape, jnp.int32)
    for j in range(1, k // _CH):
        sc = s[j * _CH:(j + 1) * _CH, :]
        lt = sc < acc_v
        acc_v = jnp.where(lt, sc, acc_v)
        acc_c = jnp.where(lt, j, acc_c)
    m = jnp.min(acc_v, axis=0, keepdims=True)
    kpos = acc_c * _CH + lax.broadcasted_iota(jnp.int32, acc_c.shape, 0)
    cand = jnp.where(acc_v == m, kpos, k)
    idx = jnp.min(cand, axis=0, keepdims=True)
    idx_ref[...] = idx.reshape(idx_ref.shape)


def _encode_and_lut(zt, w1t, b1c, w2t, b2c, cb, w3p, b3r, *, rows):
    n = zt.shape[1]
    k = cb.shape[0]
    nbp = w3p.shape[1]
    grid = (n // rows,)
    return pl.pallas_call(
        _tc_body,
        grid=grid,
        in_specs=[
            pl.BlockSpec((zt.shape[0], rows), lambda i: (0, i)),
            pl.BlockSpec(w1t.shape, lambda i: (0, 0)),
            pl.BlockSpec(b1c.shape, lambda i: (0, 0)),
            pl.BlockSpec(w2t.shape, lambda i: (0, 0)),
            pl.BlockSpec(b2c.shape, lambda i: (0, 0)),
            pl.BlockSpec(cb.shape, lambda i: (0, 0)),
            pl.BlockSpec(w3p.shape, lambda i: (0, 0)),
            pl.BlockSpec(b3r.shape, lambda i: (0, 0)),
        ],
        out_specs=[
            pl.BlockSpec((1, 1, rows), lambda i: (i, 0, 0)),
            pl.BlockSpec((k, nbp), lambda i: (0, 0)),
        ],
        out_shape=[
            jax.ShapeDtypeStruct((n // rows, 1, rows), jnp.int32),
            jax.ShapeDtypeStruct((k, nbp), jnp.float32),
        ],
        scratch_shapes=[pltpu.VMEM((k, 2 * cb.shape[1]), jnp.bfloat16)],
    )(zt, w1t, b1c, w2t, b2c, cb, w3p, b3r)


def _sc_gather(lutp, idx2, nb):
    n = idx2.shape[0] * 128
    rows_per_tile = n // 32
    idx_rows = rows_per_tile // 128

    mesh = plsc.VectorSubcoreMesh(core_axis_name="c", subcore_axis_name="s")

    @functools.partial(
        pl.kernel,
        mesh=mesh,
        compiler_params=pltpu.CompilerParams(needs_layout_passes=False),
        out_type=jax.ShapeDtypeStruct((32, nb, rows_per_tile), jnp.float32),
        scratch_types=[
            pltpu.VMEM(lutp.shape, jnp.float32),
            pltpu.VMEM((idx_rows, 128), jnp.int32),
            pltpu.VMEM((nb, rows_per_tile), jnp.float32),
            pltpu.SemaphoreType.DMA,
        ],
    )
    def k(lut_hbm, idx_hbm, out_hbm, lut_v, idx_v, out_v, sem):
        wid = lax.axis_index("s") * 2 + lax.axis_index("c")
        cp = pltpu.async_copy(lut_hbm, lut_v, sem)
        pltpu.sync_copy(idx_hbm.at[pl.ds(wid * idx_rows, idx_rows)], idx_v)
        cp.wait()

        def row_body(row, carry):
            for g in range(8):
                iv = idx_v[row, pl.ds(g * 16, 16)]
                ivd = lax.shift_right_logical(iv, 3)
                ivm = lax.shift_left(jnp.bitwise_and(iv, 7), 4)
                for j in range(nb):
                    vals = plsc.load_gather(lut_v, [ivd, ivm + j])
                    out_v[j, pl.ds(row * 128 + g * 16, 16)] = vals
            return carry

        lax.fori_loop(0, idx_rows, row_body, 0)
        pltpu.sync_copy(out_v, out_hbm.at[wid])

    return k(lutp, idx2)


def kernel(z, W1, b1, W2, b2, W3, b3, codebook):
    bsz, s, f = z.shape
    h = W1.shape[1]
    nb = W3.shape[1]
    n = bsz * s

    zt = jnp.pad(z.reshape(n, f), ((0, 0), (0, 8 - f))).T
    w1t = jnp.pad(W1, ((0, 8 - f), (0, 0))).T
    w3p = jnp.pad(W3, ((0, 0), (0, 16 - nb)))
    b3r = jnp.pad(b3, ((0, 16 - nb),)).reshape(1, 16)
    b1c = b1.reshape(h, 1)
    b2c = b2.reshape(h, 1)

    idx_n1, lutb = _encode_and_lut(zt, w1t, b1c, w2t=W2.T, b2c=b2c,
                                   cb=codebook, w3p=w3p, b3r=b3r, rows=512)
    idx2 = idx_n1.reshape(n // 128, 128)
    lutp = lutb.reshape(-1, 128)
    gathered = _sc_gather(lutp, idx2, nb)
    return gathered.transpose(0, 2, 1).reshape(bsz, s * nb)


# speedup vs baseline: 6.3028x; 1.1030x over previous
import functools

import jax
import jax.numpy as jnp
from jax import lax
from jax.experimental import pallas as pl
from jax.experimental.pallas import tpu as pltpu
from jax.experimental.pallas import tpu_sc as plsc


_CH = 16


def _tc_body(zt_ref, w1t_ref, b1c_ref, w2t_ref, b2c_ref, cb_ref,
             w3_ref, b3_ref, idx_ref, lut_ref, aug_ref):
    i = pl.program_id(0)
    k = cb_ref.shape[0]
    h = cb_ref.shape[1]

    @pl.when(i == 0)
    def _():
        cb = cb_ref[...]
        lut_ref[...] = (jnp.dot(cb, w3_ref[...],
                                preferred_element_type=jnp.float32)
                        + b3_ref[...])
        cc = jnp.sum(cb * cb, axis=1, keepdims=True)
        cc1 = cc.astype(jnp.bfloat16).astype(jnp.float32)
        r1 = cc - cc1
        cc2 = r1.astype(jnp.bfloat16).astype(jnp.float32)
        cc3 = (r1 - cc2).astype(jnp.bfloat16).astype(jnp.float32)
        lane = lax.broadcasted_iota(jnp.int32, (k, h), 1)
        hi = jnp.where(lane == 0, cc1,
                       jnp.where(lane == 1, cc2,
                                 jnp.where(lane == 2, cc3, 0.0)))
        aug_ref[:, 0:h] = cb.astype(jnp.bfloat16)
        aug_ref[:, h:2 * h] = hi.astype(jnp.bfloat16)

    h1 = jnp.dot(w1t_ref[...], zt_ref[...], preferred_element_type=jnp.float32)
    a1 = jnp.maximum(h1 + b1c_ref[...], 0.0)
    h2 = jnp.dot(w2t_ref[...], a1, preferred_element_type=jnp.float32)
    a2 = jnp.maximum(h2 + b2c_ref[...], 0.0)

    sub = lax.broadcasted_iota(jnp.int32, a2.shape, 0)
    ones3 = jnp.where(sub < 3, 1.0, 0.0)
    rhs = jnp.concatenate([a2 * -2.0, ones3], axis=0)
    s = jnp.dot(aug_ref[...], rhs.astype(jnp.bfloat16),
                preferred_element_type=jnp.float32)

    acc_v = s[0:_CH, :]
    acc_c = jnp.zeros(acc_v.shape, jnp.int32)
    for j in range(1, k // _CH):
        sc = s[j * _CH:(j + 1) * _CH, :]
        lt = sc < acc_v
        acc_v = jnp.where(lt, sc, acc_v)
        acc_c = jnp.where(lt, j, acc_c)
    m = jnp.min(acc_v, axis=0, keepdims=True)
    kpos = acc_c * _CH + lax.broadcasted_iota(jnp.int32, acc_c.shape, 0)
    cand = jnp.where(acc_v == m, kpos, k)
    idx = jnp.min(cand, axis=0, keepdims=True)
    idx_ref[...] = idx.reshape(idx_ref.shape)


def _encode_and_lut(zt, w1t, b1c, w2t, b2c, cb, w3p, b3r, *, rows):
    n = zt.shape[1]
    k = cb.shape[0]
    nbp = w3p.shape[1]
    grid = (n // rows,)
    return pl.pallas_call(
        _tc_body,
        grid=grid,
        in_specs=[
            pl.BlockSpec((zt.shape[0], rows), lambda i: (0, i)),
            pl.BlockSpec(w1t.shape, lambda i: (0, 0)),
            pl.BlockSpec(b1c.shape, lambda i: (0, 0)),
            pl.BlockSpec(w2t.shape, lambda i: (0, 0)),
            pl.BlockSpec(b2c.shape, lambda i: (0, 0)),
            pl.BlockSpec(cb.shape, lambda i: (0, 0)),
            pl.BlockSpec(w3p.shape, lambda i: (0, 0)),
            pl.BlockSpec(b3r.shape, lambda i: (0, 0)),
        ],
        out_specs=[
            pl.BlockSpec((1, 1, rows), lambda i: (i, 0, 0)),
            pl.BlockSpec((k, nbp), lambda i: (0, 0)),
        ],
        out_shape=[
            jax.ShapeDtypeStruct((n // rows, 1, rows), jnp.int32),
            jax.ShapeDtypeStruct((k, nbp), jnp.float32),
        ],
        scratch_shapes=[pltpu.VMEM((k, 2 * cb.shape[1]), jnp.bfloat16)],
    )(zt, w1t, b1c, w2t, b2c, cb, w3p, b3r)


def _sc_gather(lutp, idx2, nb):
    n = idx2.shape[0] * 128
    rows_per_tile = n // 32
    idx_rows = rows_per_tile // 128

    mesh = plsc.VectorSubcoreMesh(core_axis_name="c", subcore_axis_name="s")

    @functools.partial(
        pl.kernel,
        mesh=mesh,
        compiler_params=pltpu.CompilerParams(needs_layout_passes=False),
        out_type=jax.ShapeDtypeStruct((32, nb, rows_per_tile), jnp.float32),
        scratch_types=[
            pltpu.VMEM(lutp.shape, jnp.float32),
            pltpu.VMEM((idx_rows, 128), jnp.int32),
            pltpu.VMEM((nb, rows_per_tile), jnp.float32),
            pltpu.SemaphoreType.DMA,
        ],
    )
    def k(lut_hbm, idx_hbm, out_hbm, lut_v, idx_v, out_v, sem):
        wid = lax.axis_index("s") * 2 + lax.axis_index("c")
        cp = pltpu.async_copy(lut_hbm, lut_v, sem)
        pltpu.sync_copy(idx_hbm.at[pl.ds(wid * idx_rows, idx_rows)], idx_v)
        cp.wait()

        def row_body(row, carry):
            for g in range(8):
                iv = idx_v[row, pl.ds(g * 16, 16)]
                ivd = lax.shift_right_logical(iv, 3)
                ivm = lax.shift_left(jnp.bitwise_and(iv, 7), 4)
                for j in range(nb):
                    vals = plsc.load_gather(lut_v, [ivd, ivm + j])
                    out_v[j, pl.ds(row * 128 + g * 16, 16)] = vals
            return carry

        lax.fori_loop(0, idx_rows, row_body, 0)
        pltpu.sync_copy(out_v, out_hbm.at[wid])

    return k(lutp, idx2)


def kernel(z, W1, b1, W2, b2, W3, b3, codebook):
    bsz, s, f = z.shape
    h = W1.shape[1]
    nb = W3.shape[1]
    n = bsz * s

    zt = jnp.pad(z.reshape(n, f), ((0, 0), (0, 8 - f))).T
    w1t = jnp.pad(W1, ((0, 8 - f), (0, 0))).T
    w3p = jnp.pad(W3, ((0, 0), (0, 16 - nb)))
    b3r = jnp.pad(b3, ((0, 16 - nb),)).reshape(1, 16)
    b1c = b1.reshape(h, 1)
    b2c = b2.reshape(h, 1)

    idx_n1, lutb = _encode_and_lut(zt, w1t, b1c, w2t=W2.T, b2c=b2c,
                                   cb=codebook, w3p=w3p, b3r=b3r, rows=1024)
    idx2 = idx_n1.reshape(n // 128, 128)
    lutp = lutb.reshape(-1, 128)
    gathered = _sc_gather(lutp, idx2, nb)
    return gathered.transpose(0, 2, 1).reshape(bsz, s * nb)


# speedup vs baseline: 6.6791x; 1.0597x over previous
import functools

import jax
import jax.numpy as jnp
from jax import lax
from jax.experimental import pallas as pl
from jax.experimental.pallas import tpu as pltpu
from jax.experimental.pallas import tpu_sc as plsc


_CH = 16


def _tc_body(zt_ref, w1t_ref, b1c_ref, w2t_ref, b2c_ref, cb_ref,
             w3_ref, b3_ref, idx_ref, lut_ref, aug_ref):
    i = pl.program_id(0)
    k = cb_ref.shape[0]
    h = cb_ref.shape[1]

    @pl.when(i == 0)
    def _():
        cb = cb_ref[...]
        lut_ref[...] = (jnp.dot(cb, w3_ref[...],
                                preferred_element_type=jnp.float32)
                        + b3_ref[...])
        cc = jnp.sum(cb * cb, axis=1, keepdims=True)
        cc1 = cc.astype(jnp.bfloat16).astype(jnp.float32)
        r1 = cc - cc1
        cc2 = r1.astype(jnp.bfloat16).astype(jnp.float32)
        cc3 = (r1 - cc2).astype(jnp.bfloat16).astype(jnp.float32)
        lane = lax.broadcasted_iota(jnp.int32, (k, h), 1)
        hi = jnp.where(lane == 0, cc1,
                       jnp.where(lane == 1, cc2,
                                 jnp.where(lane == 2, cc3, 0.0)))
        aug_ref[:, 0:h] = cb.astype(jnp.bfloat16)
        aug_ref[:, h:2 * h] = hi.astype(jnp.bfloat16)

    h1 = jnp.dot(w1t_ref[...], zt_ref[...], preferred_element_type=jnp.float32)
    a1 = jnp.maximum(h1 + b1c_ref[...], 0.0)
    h2 = jnp.dot(w2t_ref[...], a1, preferred_element_type=jnp.float32)
    a2 = jnp.maximum(h2 + b2c_ref[...], 0.0)

    sub = lax.broadcasted_iota(jnp.int32, a2.shape, 0)
    ones3 = jnp.where(sub < 3, 1.0, 0.0)
    rhs = jnp.concatenate([a2 * -2.0, ones3], axis=0)
    s = jnp.dot(aug_ref[...], rhs.astype(jnp.bfloat16),
                preferred_element_type=jnp.float32)

    acc_v = s[0:_CH, :]
    acc_c = jnp.zeros(acc_v.shape, jnp.int32)
    for j in range(1, k // _CH):
        sc = s[j * _CH:(j + 1) * _CH, :]
        lt = sc < acc_v
        acc_v = jnp.where(lt, sc, acc_v)
        acc_c = jnp.where(lt, j, acc_c)
    m = jnp.min(acc_v, axis=0, keepdims=True)
    kpos = acc_c * _CH + lax.broadcasted_iota(jnp.int32, acc_c.shape, 0)
    cand = jnp.where(acc_v == m, kpos, k)
    idx = jnp.min(cand, axis=0, keepdims=True)
    idx_ref[...] = idx.reshape(idx_ref.shape)


def _encode_and_lut(zt, w1t, b1c, w2t, b2c, cb, w3p, b3r, *, rows):
    n = zt.shape[1]
    k = cb.shape[0]
    nbp = w3p.shape[1]
    grid = (n // rows,)
    return pl.pallas_call(
        _tc_body,
        grid=grid,
        in_specs=[
            pl.BlockSpec((zt.shape[0], rows), lambda i: (0, i)),
            pl.BlockSpec(w1t.shape, lambda i: (0, 0)),
            pl.BlockSpec(b1c.shape, lambda i: (0, 0)),
            pl.BlockSpec(w2t.shape, lambda i: (0, 0)),
            pl.BlockSpec(b2c.shape, lambda i: (0, 0)),
            pl.BlockSpec(cb.shape, lambda i: (0, 0)),
            pl.BlockSpec(w3p.shape, lambda i: (0, 0)),
            pl.BlockSpec(b3r.shape, lambda i: (0, 0)),
        ],
        out_specs=[
            pl.BlockSpec((1, 1, rows), lambda i: (i, 0, 0)),
            pl.BlockSpec((k, nbp), lambda i: (0, 0)),
        ],
        out_shape=[
            jax.ShapeDtypeStruct((n // rows, 1, rows), jnp.int32),
            jax.ShapeDtypeStruct((k, nbp), jnp.float32),
        ],
        scratch_shapes=[pltpu.VMEM((k, 2 * cb.shape[1]), jnp.bfloat16)],
    )(zt, w1t, b1c, w2t, b2c, cb, w3p, b3r)


def _sc_gather(lutp, idx2, nb):
    n = idx2.shape[0] * 128
    rows_per_tile = n // 32
    idx_rows = rows_per_tile // 128

    mesh = plsc.VectorSubcoreMesh(core_axis_name="c", subcore_axis_name="s")

    @functools.partial(
        pl.kernel,
        mesh=mesh,
        compiler_params=pltpu.CompilerParams(needs_layout_passes=False),
        out_type=jax.ShapeDtypeStruct((32, nb, rows_per_tile), jnp.float32),
        scratch_types=[
            pltpu.VMEM(lutp.shape, jnp.float32),
            pltpu.VMEM((idx_rows, 128), jnp.int32),
            pltpu.VMEM((nb, rows_per_tile), jnp.float32),
            pltpu.SemaphoreType.DMA,
        ],
    )
    def k(lut_hbm, idx_hbm, out_hbm, lut_v, idx_v, out_v, sem):
        wid = lax.axis_index("s") * 2 + lax.axis_index("c")
        cp = pltpu.async_copy(lut_hbm, lut_v, sem)
        pltpu.sync_copy(idx_hbm.at[pl.ds(wid * idx_rows, idx_rows)], idx_v)
        cp.wait()

        def row_body(row, carry):
            for g in range(8):
                iv = idx_v[row, pl.ds(g * 16, 16)]
                ivd = lax.shift_right_logical(iv, 3)
                ivm = lax.shift_left(jnp.bitwise_and(iv, 7), 4)
                for j in range(nb):
                    vals = plsc.load_gather(lut_v, [ivd, ivm + j])
                    out_v[j, pl.ds(row * 128 + g * 16, 16)] = vals
            return carry

        lax.fori_loop(0, idx_rows, row_body, 0)
        pltpu.sync_copy(out_v, out_hbm.at[wid])

    return k(lutp, idx2)


def kernel(z, W1, b1, W2, b2, W3, b3, codebook):
    bsz, s, f = z.shape
    h = W1.shape[1]
    nb = W3.shape[1]
    n = bsz * s

    zt = jnp.pad(z.reshape(n, f), ((0, 0), (0, 8 - f))).T
    w1t = jnp.pad(W1, ((0, 8 - f), (0, 0))).T
    w3p = jnp.pad(W3, ((0, 0), (0, 16 - nb)))
    b3r = jnp.pad(b3, ((0, 16 - nb),)).reshape(1, 16)
    b1c = b1.reshape(h, 1)
    b2c = b2.reshape(h, 1)

    idx_n1, lutb = _encode_and_lut(zt, w1t, b1c, w2t=W2.T, b2c=b2c,
                                   cb=codebook, w3p=w3p, b3r=b3r, rows=2048)
    idx2 = idx_n1.reshape(n // 128, 128)
    lutp = lutb.reshape(-1, 128)
    gathered = _sc_gather(lutp, idx2, nb)
    return gathered.transpose(0, 2, 1).reshape(bsz, s * nb)


# speedup vs baseline: 6.6804x; 1.0002x over previous
import functools

import jax
import jax.numpy as jnp
from jax import lax
from jax.experimental import pallas as pl
from jax.experimental.pallas import tpu as pltpu
from jax.experimental.pallas import tpu_sc as plsc


_CH = 16


def _tc_body(zt_ref, w1t_ref, b1c_ref, w2t_ref, b2c_ref, cb_ref,
             w3_ref, b3_ref, idx_ref, lut_ref, aug_ref):
    i = pl.program_id(0)
    k = cb_ref.shape[0]
    h = cb_ref.shape[1]

    @pl.when(i == 0)
    def _():
        cb = cb_ref[...]
        lut_ref[...] = (jnp.dot(cb, w3_ref[...],
                                preferred_element_type=jnp.float32)
                        + b3_ref[...])
        cc = jnp.sum(cb * cb, axis=1, keepdims=True)
        cc1 = cc.astype(jnp.bfloat16).astype(jnp.float32)
        r1 = cc - cc1
        cc2 = r1.astype(jnp.bfloat16).astype(jnp.float32)
        cc3 = (r1 - cc2).astype(jnp.bfloat16).astype(jnp.float32)
        lane = lax.broadcasted_iota(jnp.int32, (k, h), 1)
        hi = jnp.where(lane == 0, cc1,
                       jnp.where(lane == 1, cc2,
                                 jnp.where(lane == 2, cc3, 0.0)))
        aug_ref[:, 0:h] = cb.astype(jnp.bfloat16)
        aug_ref[:, h:2 * h] = hi.astype(jnp.bfloat16)

    h1 = jnp.dot(w1t_ref[...], zt_ref[...], preferred_element_type=jnp.float32)
    a1 = jnp.maximum(h1 + b1c_ref[...], 0.0)
    h2 = jnp.dot(w2t_ref[...], a1, preferred_element_type=jnp.float32)
    a2 = jnp.maximum(h2 + b2c_ref[...], 0.0)

    sub = lax.broadcasted_iota(jnp.int32, a2.shape, 0)
    ones3 = jnp.where(sub < 3, 1.0, 0.0)
    rhs = jnp.concatenate([a2 * -2.0, ones3], axis=0)
    s = jnp.dot(aug_ref[...], rhs.astype(jnp.bfloat16),
                preferred_element_type=jnp.float32)

    acc_v = s[0:_CH, :]
    acc_c = jnp.zeros(acc_v.shape, jnp.int32)
    for j in range(1, k // _CH):
        sc = s[j * _CH:(j + 1) * _CH, :]
        lt = sc < acc_v
        acc_v = jnp.minimum(acc_v, sc)
        acc_c = jnp.where(lt, j, acc_c)
    m = jnp.min(acc_v, axis=0, keepdims=True)
    kpos = acc_c * _CH + lax.broadcasted_iota(jnp.int32, acc_c.shape, 0)
    cand = jnp.where(acc_v == m, kpos, k)
    idx = jnp.min(cand, axis=0, keepdims=True)
    idx_ref[...] = idx.reshape(idx_ref.shape)


def _encode_and_lut(zt, w1t, b1c, w2t, b2c, cb, w3p, b3r, *, rows):
    n = zt.shape[1]
    k = cb.shape[0]
    nbp = w3p.shape[1]
    grid = (n // rows,)
    return pl.pallas_call(
        _tc_body,
        grid=grid,
        in_specs=[
            pl.BlockSpec((zt.shape[0], rows), lambda i: (0, i)),
            pl.BlockSpec(w1t.shape, lambda i: (0, 0)),
            pl.BlockSpec(b1c.shape, lambda i: (0, 0)),
            pl.BlockSpec(w2t.shape, lambda i: (0, 0)),
            pl.BlockSpec(b2c.shape, lambda i: (0, 0)),
            pl.BlockSpec(cb.shape, lambda i: (0, 0)),
            pl.BlockSpec(w3p.shape, lambda i: (0, 0)),
            pl.BlockSpec(b3r.shape, lambda i: (0, 0)),
        ],
        out_specs=[
            pl.BlockSpec((1, 1, rows), lambda i: (i, 0, 0)),
            pl.BlockSpec((k, nbp), lambda i: (0, 0)),
        ],
        out_shape=[
            jax.ShapeDtypeStruct((n // rows, 1, rows), jnp.int32),
            jax.ShapeDtypeStruct((k, nbp), jnp.float32),
        ],
        scratch_shapes=[pltpu.VMEM((k, 2 * cb.shape[1]), jnp.bfloat16)],
    )(zt, w1t, b1c, w2t, b2c, cb, w3p, b3r)


def _sc_gather(lutp, idx2, nb):
    n = idx2.shape[0] * 128
    rows_per_tile = n // 32
    idx_rows = rows_per_tile // 128

    mesh = plsc.VectorSubcoreMesh(core_axis_name="c", subcore_axis_name="s")

    @functools.partial(
        pl.kernel,
        mesh=mesh,
        compiler_params=pltpu.CompilerParams(needs_layout_passes=False),
        out_type=jax.ShapeDtypeStruct((32, nb, rows_per_tile), jnp.float32),
        scratch_types=[
            pltpu.VMEM(lutp.shape, jnp.float32),
            pltpu.VMEM((idx_rows, 128), jnp.int32),
            pltpu.VMEM((nb, rows_per_tile), jnp.float32),
            pltpu.SemaphoreType.DMA,
        ],
    )
    def k(lut_hbm, idx_hbm, out_hbm, lut_v, idx_v, out_v, sem):
        wid = lax.axis_index("s") * 2 + lax.axis_index("c")
        cp = pltpu.async_copy(lut_hbm, lut_v, sem)
        pltpu.sync_copy(idx_hbm.at[pl.ds(wid * idx_rows, idx_rows)], idx_v)
        cp.wait()

        def row_body(row, carry):
            for g in range(8):
                iv = idx_v[row, pl.ds(g * 16, 16)]
                ivd = lax.shift_right_logical(iv, 3)
                ivm = lax.shift_left(jnp.bitwise_and(iv, 7), 4)
                for j in range(nb):
                    vals = plsc.load_gather(lut_v, [ivd, ivm + j])
                    out_v[j, pl.ds(row * 128 + g * 16, 16)] = vals
            return carry

        lax.fori_loop(0, idx_rows, row_body, 0)
        pltpu.sync_copy(out_v, out_hbm.at[wid])

    return k(lutp, idx2)


def kernel(z, W1, b1, W2, b2, W3, b3, codebook):
    bsz, s, f = z.shape
    h = W1.shape[1]
    nb = W3.shape[1]
    n = bsz * s

    zt = jnp.pad(z.reshape(n, f), ((0, 0), (0, 8 - f))).T
    w1t = jnp.pad(W1, ((0, 8 - f), (0, 0))).T
    w3p = jnp.pad(W3, ((0, 0), (0, 16 - nb)))
    b3r = jnp.pad(b3, ((0, 16 - nb),)).reshape(1, 16)
    b1c = b1.reshape(h, 1)
    b2c = b2.reshape(h, 1)

    idx_n1, lutb = _encode_and_lut(zt, w1t, b1c, w2t=W2.T, b2c=b2c,
                                   cb=codebook, w3p=w3p, b3r=b3r, rows=2048)
    idx2 = idx_n1.reshape(n // 128, 128)
    lutp = lutb.reshape(-1, 128)
    gathered = _sc_gather(lutp, idx2, nb)
    return gathered.transpose(0, 2, 1).reshape(bsz, s * nb)


# speedup vs baseline: 7.9466x; 1.1895x over previous
import functools

import jax
import jax.numpy as jnp
from jax import lax
from jax.experimental import pallas as pl
from jax.experimental.pallas import tpu as pltpu
from jax.experimental.pallas import tpu_sc as plsc


_CH = 16


def _tc_body(zt_ref, w1t_ref, b1c_ref, w2t_ref, b2c_ref, cb_ref,
             w3_ref, b3_ref, idx_ref, lut_ref, aug_ref):
    i = pl.program_id(0)
    k = cb_ref.shape[0]
    h = cb_ref.shape[1]

    @pl.when(i == 0)
    def _():
        cb = cb_ref[...]
        lut_ref[...] = (jnp.dot(cb, w3_ref[...],
                                preferred_element_type=jnp.float32)
                        + b3_ref[...])
        cc = jnp.sum(cb * cb, axis=1, keepdims=True)
        cc1 = cc.astype(jnp.bfloat16).astype(jnp.float32)
        r1 = cc - cc1
        cc2 = r1.astype(jnp.bfloat16).astype(jnp.float32)
        cc3 = (r1 - cc2).astype(jnp.bfloat16).astype(jnp.float32)
        lane = lax.broadcasted_iota(jnp.int32, (k, h), 1)
        hi = jnp.where(lane == 0, cc1,
                       jnp.where(lane == 1, cc2,
                                 jnp.where(lane == 2, cc3, 0.0)))
        aug_ref[:, 0:h] = cb.astype(jnp.bfloat16)
        aug_ref[:, h:2 * h] = hi.astype(jnp.bfloat16)

    h1 = jnp.dot(w1t_ref[...], zt_ref[...], preferred_element_type=jnp.float32)
    a1 = jnp.maximum(h1 + b1c_ref[...], 0.0)
    h2 = jnp.dot(w2t_ref[...], a1, preferred_element_type=jnp.float32)
    a2 = jnp.maximum(h2 + b2c_ref[...], 0.0)

    sub = lax.broadcasted_iota(jnp.int32, a2.shape, 0)
    ones3 = jnp.where(sub < 3, 1.0, 0.0)
    rhs = jnp.concatenate([a2 * -2.0, ones3], axis=0)
    s = jnp.dot(aug_ref[...], rhs.astype(jnp.bfloat16),
                preferred_element_type=jnp.float32)

    acc_v = s[0:_CH, :]
    acc_c = jnp.zeros(acc_v.shape, jnp.int32)
    for j in range(1, k // _CH):
        sc = s[j * _CH:(j + 1) * _CH, :]
        lt = sc < acc_v
        acc_v = jnp.minimum(acc_v, sc)
        acc_c = jnp.where(lt, j, acc_c)
    m = jnp.min(acc_v, axis=0, keepdims=True)
    kpos = acc_c * _CH + lax.broadcasted_iota(jnp.int32, acc_c.shape, 0)
    cand = jnp.where(acc_v == m, kpos, k)
    idx = jnp.min(cand, axis=0, keepdims=True)
    idx_ref[...] = idx.reshape(idx_ref.shape)


def _encode_and_lut(zt, w1t, b1c, w2t, b2c, cb, w3p, b3r, *, rows):
    n = zt.shape[1]
    k = cb.shape[0]
    nbp = w3p.shape[1]
    grid = (n // rows,)
    return pl.pallas_call(
        _tc_body,
        grid=grid,
        in_specs=[
            pl.BlockSpec((zt.shape[0], rows), lambda i: (0, i)),
            pl.BlockSpec(w1t.shape, lambda i: (0, 0)),
            pl.BlockSpec(b1c.shape, lambda i: (0, 0)),
            pl.BlockSpec(w2t.shape, lambda i: (0, 0)),
            pl.BlockSpec(b2c.shape, lambda i: (0, 0)),
            pl.BlockSpec(cb.shape, lambda i: (0, 0)),
            pl.BlockSpec(w3p.shape, lambda i: (0, 0)),
            pl.BlockSpec(b3r.shape, lambda i: (0, 0)),
        ],
        out_specs=[
            pl.BlockSpec((1, 1, rows), lambda i: (i, 0, 0)),
            pl.BlockSpec((k, nbp), lambda i: (0, 0)),
        ],
        out_shape=[
            jax.ShapeDtypeStruct((n // rows, 1, rows), jnp.int32),
            jax.ShapeDtypeStruct((k, nbp), jnp.float32),
        ],
        scratch_shapes=[pltpu.VMEM((k, 2 * cb.shape[1]), jnp.bfloat16)],
    )(zt, w1t, b1c, w2t, b2c, cb, w3p, b3r)


def _sc_gather(lutp, idx2, nb, sym):
    n = idx2.shape[0] * 128
    rows_per_tile = n // 32
    idx_rows = rows_per_tile // 128
    batches_per_tile = rows_per_tile // sym
    cols = sym * nb

    mesh = plsc.VectorSubcoreMesh(core_axis_name="c", subcore_axis_name="s")

    @functools.partial(
        pl.kernel,
        mesh=mesh,
        compiler_params=pltpu.CompilerParams(needs_layout_passes=False),
        out_type=jax.ShapeDtypeStruct((n // sym, cols), jnp.float32),
        scratch_types=[
            pltpu.VMEM(lutp.shape, jnp.float32),
            pltpu.VMEM((idx_rows, 128), jnp.int32),
            pltpu.VMEM((batches_per_tile, cols), jnp.float32),
            pltpu.SemaphoreType.DMA,
        ],
    )
    def k(lut_hbm, idx_hbm, out_hbm, lut_v, idx_v, out_v, sem):
        wid = lax.axis_index("s") * 2 + lax.axis_index("c")
        cp = pltpu.async_copy(lut_hbm, lut_v, sem)
        pltpu.sync_copy(idx_hbm.at[pl.ds(wid * idx_rows, idx_rows)], idx_v)
        cp.wait()
        lane6 = lax.iota(jnp.int32, 16) * nb

        def row_body(row, carry):
            for g in range(8):
                base = row * 128 + g * 16
                iv = idx_v[row, pl.ds(g * 16, 16)]
                ivd = lax.shift_right_logical(iv, 3)
                ivm = lax.shift_left(jnp.bitwise_and(iv, 7), 4)
                brow = lax.div(base, sym)
                colb = lax.rem(base, sym) * nb
                idx0 = jnp.full((16,), brow, jnp.int32)
                for j in range(nb):
                    vals = plsc.load_gather(lut_v, [ivd, ivm + j])
                    plsc.store_scatter(out_v, [idx0, lane6 + (colb + j)], vals)
            return carry

        lax.fori_loop(0, idx_rows, row_body, 0)
        pltpu.sync_copy(out_v,
                        out_hbm.at[pl.ds(wid * batches_per_tile,
                                         batches_per_tile)])

    return k(lutp, idx2)


def kernel(z, W1, b1, W2, b2, W3, b3, codebook):
    bsz, s, f = z.shape
    h = W1.shape[1]
    nb = W3.shape[1]
    n = bsz * s

    zt = jnp.pad(z.reshape(n, f), ((0, 0), (0, 8 - f))).T
    w1t = jnp.pad(W1, ((0, 8 - f), (0, 0))).T
    w3p = jnp.pad(W3, ((0, 0), (0, 16 - nb)))
    b3r = jnp.pad(b3, ((0, 16 - nb),)).reshape(1, 16)
    b1c = b1.reshape(h, 1)
    b2c = b2.reshape(h, 1)

    idx_n1, lutb = _encode_and_lut(zt, w1t, b1c, w2t=W2.T, b2c=b2c,
                                   cb=codebook, w3p=w3p, b3r=b3r, rows=2048)
    idx2 = idx_n1.reshape(n // 128, 128)
    lutp = lutb.reshape(-1, 128)
    return _sc_gather(lutp, idx2, nb, s)


# speedup vs baseline: 7.9500x; 1.0004x over previous
import functools

import jax
import jax.numpy as jnp
from jax import lax
from jax.experimental import pallas as pl
from jax.experimental.pallas import tpu as pltpu
from jax.experimental.pallas import tpu_sc as plsc


_CH = 16


def _tc_body(zt_ref, w1t_ref, b1c_ref, w2t_ref, b2c_ref, cb_ref,
             w3_ref, b3_ref, idx_ref, lut_ref, aug_ref):
    i = pl.program_id(0)
    k = cb_ref.shape[0]
    h = cb_ref.shape[1]

    @pl.when(i == 0)
    def _():
        cb = cb_ref[...]
        lut_ref[...] = (jnp.dot(cb, w3_ref[...],
                                preferred_element_type=jnp.float32)
                        + b3_ref[...])
        cc = jnp.sum(cb * cb, axis=1, keepdims=True)
        cc1 = cc.astype(jnp.bfloat16).astype(jnp.float32)
        r1 = cc - cc1
        cc2 = r1.astype(jnp.bfloat16).astype(jnp.float32)
        cc3 = (r1 - cc2).astype(jnp.bfloat16).astype(jnp.float32)
        lane = lax.broadcasted_iota(jnp.int32, (k, h), 1)
        hi = jnp.where(lane == 0, cc1,
                       jnp.where(lane == 1, cc2,
                                 jnp.where(lane == 2, cc3, 0.0)))
        aug_ref[:, 0:h] = cb.astype(jnp.bfloat16)
        aug_ref[:, h:2 * h] = hi.astype(jnp.bfloat16)

    h1 = jnp.dot(w1t_ref[...], zt_ref[...], preferred_element_type=jnp.float32)
    a1 = jnp.maximum(h1 + b1c_ref[...], 0.0)
    h2 = jnp.dot(w2t_ref[...], a1, preferred_element_type=jnp.float32)
    a2 = jnp.maximum(h2 + b2c_ref[...], 0.0)

    sub = lax.broadcasted_iota(jnp.int32, a2.shape, 0)
    ones3 = jnp.where(sub < 3, 1.0, 0.0)
    rhs = jnp.concatenate([a2 * -2.0, ones3], axis=0)
    s = jnp.dot(aug_ref[...], rhs.astype(jnp.bfloat16),
                preferred_element_type=jnp.float32)

    acc_v = s[0:_CH, :]
    acc_c = jnp.zeros(acc_v.shape, jnp.int32)
    for j in range(1, k // _CH):
        sc = s[j * _CH:(j + 1) * _CH, :]
        lt = sc < acc_v
        acc_v = jnp.minimum(acc_v, sc)
        acc_c = jnp.where(lt, j, acc_c)
    m = jnp.min(acc_v, axis=0, keepdims=True)
    kpos = acc_c * _CH + lax.broadcasted_iota(jnp.int32, acc_c.shape, 0)
    cand = jnp.where(acc_v == m, kpos, k)
    idx = jnp.min(cand, axis=0, keepdims=True)
    idx_ref[...] = idx.reshape(idx_ref.shape)


def _encode_and_lut(zt, w1t, b1c, w2t, b2c, cb, w3p, b3r, *, rows):
    n = zt.shape[1]
    k = cb.shape[0]
    nbp = w3p.shape[1]
    grid = (n // rows,)
    return pl.pallas_call(
        _tc_body,
        grid=grid,
        in_specs=[
            pl.BlockSpec((zt.shape[0], rows), lambda i: (0, i)),
            pl.BlockSpec(w1t.shape, lambda i: (0, 0)),
            pl.BlockSpec(b1c.shape, lambda i: (0, 0)),
            pl.BlockSpec(w2t.shape, lambda i: (0, 0)),
            pl.BlockSpec(b2c.shape, lambda i: (0, 0)),
            pl.BlockSpec(cb.shape, lambda i: (0, 0)),
            pl.BlockSpec(w3p.shape, lambda i: (0, 0)),
            pl.BlockSpec(b3r.shape, lambda i: (0, 0)),
        ],
        out_specs=[
            pl.BlockSpec((rows,), lambda i: (i,)),
            pl.BlockSpec((k, nbp), lambda i: (0, 0)),
        ],
        out_shape=[
            jax.ShapeDtypeStruct((n,), jnp.int32),
            jax.ShapeDtypeStruct((k, nbp), jnp.float32),
        ],
        scratch_shapes=[pltpu.VMEM((k, 2 * cb.shape[1]), jnp.bfloat16)],
    )(zt, w1t, b1c, w2t, b2c, cb, w3p, b3r)


def _sc_gather(lutp, idx1, nb, sym):
    n = idx1.shape[0]
    rows_per_tile = n // 32
    batches_per_tile = rows_per_tile // sym
    cols = sym * nb

    mesh = plsc.VectorSubcoreMesh(core_axis_name="c", subcore_axis_name="s")

    @functools.partial(
        pl.kernel,
        mesh=mesh,
        compiler_params=pltpu.CompilerParams(needs_layout_passes=False),
        out_type=jax.ShapeDtypeStruct((n // sym, cols), jnp.float32),
        scratch_types=[
            pltpu.VMEM(lutp.shape, jnp.float32),
            pltpu.VMEM((rows_per_tile,), jnp.int32),
            pltpu.VMEM((batches_per_tile, cols), jnp.float32),
            pltpu.SemaphoreType.DMA,
        ],
    )
    def k(lut_hbm, idx_hbm, out_hbm, lut_v, idx_v, out_v, sem):
        wid = lax.axis_index("s") * 2 + lax.axis_index("c")
        cp = pltpu.async_copy(lut_hbm, lut_v, sem)
        pltpu.sync_copy(idx_hbm.at[pl.ds(wid * rows_per_tile, rows_per_tile)],
                        idx_v)
        cp.wait()
        lane6 = lax.iota(jnp.int32, 16) * nb

        def row_body(row, carry):
            for g in range(8):
                base = row * 128 + g * 16
                iv = idx_v[pl.ds(base, 16)]
                ivd = lax.shift_right_logical(iv, 3)
                ivm = lax.shift_left(jnp.bitwise_and(iv, 7), 4)
                brow = lax.div(base, sym)
                colb = lax.rem(base, sym) * nb
                idx0 = jnp.full((16,), brow, jnp.int32)
                for j in range(nb):
                    vals = plsc.load_gather(lut_v, [ivd, ivm + j])
                    plsc.store_scatter(out_v, [idx0, lane6 + (colb + j)], vals)
            return carry

        lax.fori_loop(0, rows_per_tile // 128, row_body, 0)
        pltpu.sync_copy(out_v,
                        out_hbm.at[pl.ds(wid * batches_per_tile,
                                         batches_per_tile)])

    return k(lutp, idx1)


def kernel(z, W1, b1, W2, b2, W3, b3, codebook):
    bsz, s, f = z.shape
    h = W1.shape[1]
    nb = W3.shape[1]
    n = bsz * s

    zt = jnp.pad(z.reshape(n, f), ((0, 0), (0, 8 - f))).T
    w1t = jnp.pad(W1, ((0, 8 - f), (0, 0))).T
    w3p = jnp.pad(W3, ((0, 0), (0, 16 - nb)))
    b3r = jnp.pad(b3, ((0, 16 - nb),)).reshape(1, 16)
    b1c = b1.reshape(h, 1)
    b2c = b2.reshape(h, 1)

    idx1, lutb = _encode_and_lut(zt, w1t, b1c, w2t=W2.T, b2c=b2c,
                                 cb=codebook, w3p=w3p, b3r=b3r, rows=2048)
    lutp = lutb.reshape(-1, 128)
    return _sc_gather(lutp, idx1, nb, s)
